# baseline (speedup 1.0000x reference)
import jax
import jax.numpy as jnp
from jax.experimental import pallas as pl

_TOPK = 2
_BLOCK_ROWS = 1024


def _scale_kernel(x_ref, o_ref):
    o_ref[...] = x_ref[...] * jnp.float32(_TOPK)


@jax.jit
def kernel(input, exp_indices):
    T, H = input.shape
    del exp_indices
    grid = (T // _BLOCK_ROWS,)
    return pl.pallas_call(
        _scale_kernel,
        grid=grid,
        in_specs=[pl.BlockSpec((_BLOCK_ROWS, H), lambda i: (i, 0))],
        out_specs=pl.BlockSpec((_BLOCK_ROWS, H), lambda i: (i, 0)),
        out_shape=jax.ShapeDtypeStruct((T, H), input.dtype),
    )(input)


# speedup vs baseline: 1.0364x; 1.0364x over previous
import jax
import jax.numpy as jnp
from jax.experimental import pallas as pl

_TOPK = 2
_BLOCK_ROWS = 2048


def _scale_kernel(x_ref, o_ref):
    o_ref[...] = x_ref[...] * jnp.float32(_TOPK)


@jax.jit
def kernel(input, exp_indices):
    T, H = input.shape
    del exp_indices
    grid = (T // _BLOCK_ROWS,)
    return pl.pallas_call(
        _scale_kernel,
        grid=grid,
        in_specs=[pl.BlockSpec((_BLOCK_ROWS, H), lambda i: (i, 0))],
        out_specs=pl.BlockSpec((_BLOCK_ROWS, H), lambda i: (i, 0)),
        out_shape=jax.ShapeDtypeStruct((T, H), input.dtype),
    )(input)


# speedup vs baseline: 1.0640x; 1.0266x over previous
import jax
import jax.numpy as jnp
from jax.experimental import pallas as pl

_TOPK = 2
_BLOCK_ROWS = 3584


def _scale_kernel(x_ref, o_ref):
    o_ref[...] = x_ref[...] * jnp.float32(_TOPK)


@jax.jit
def kernel(input, exp_indices):
    T, H = input.shape
    del exp_indices
    grid = (pl.cdiv(T, _BLOCK_ROWS),)
    return pl.pallas_call(
        _scale_kernel,
        grid=grid,
        in_specs=[pl.BlockSpec((_BLOCK_ROWS, H), lambda i: (i, 0))],
        out_specs=pl.BlockSpec((_BLOCK_ROWS, H), lambda i: (i, 0)),
        out_shape=jax.ShapeDtypeStruct((T, H), input.dtype),
    )(input)


# speedup vs baseline: 1.0648x; 1.0008x over previous
import jax
import jax.numpy as jnp
from jax.experimental import pallas as pl

_TOPK = 2
_BLOCK_ROWS = 3712


def _scale_kernel(x_ref, o_ref):
    o_ref[...] = x_ref[...] * jnp.float32(_TOPK)


@jax.jit
def kernel(input, exp_indices):
    T, H = input.shape
    del exp_indices
    grid = (pl.cdiv(T, _BLOCK_ROWS),)
    return pl.pallas_call(
        _scale_kernel,
        grid=grid,
        in_specs=[pl.BlockSpec((_BLOCK_ROWS, H), lambda i: (i, 0))],
        out_specs=pl.BlockSpec((_BLOCK_ROWS, H), lambda i: (i, 0)),
        out_shape=jax.ShapeDtypeStruct((T, H), input.dtype),
    )(input)
